# baseline (speedup 1.0000x reference)
import functools

import jax
import jax.numpy as jnp
from jax import lax
from jax.experimental import pallas as pl
from jax.experimental.pallas import tpu as pltpu
from jax.experimental.pallas import tpu_sc as plsc

N = 10000
NP = 10240
E = 160000
EP = 163840
NSUB = 16
NCORE = 2
BLK = 128
NB = EP // (NSUB * BLK)
ZC = NP // (NSUB * 128)
GB = 64
NGB = EP // (NSUB * GB)
ZG = NP // (NSUB * GB)
RB = 256
NR = NP // RB

_mesh = plsc.VectorSubcoreMesh(core_axis_name="c", subcore_axis_name="s")


def _make_agg(C):
    CPC = C // NCORE

    @functools.partial(
        pl.kernel,
        mesh=_mesh,
        out_type=jax.ShapeDtypeStruct((C, NP, 128), jnp.float32),
        scratch_types=[
            pltpu.VMEM((NGB // 4, GB), jnp.int32),
            pltpu.VMEM((NGB // 4, GB), jnp.int32),
            pltpu.VMEM((GB, 128), jnp.float32),
            pltpu.VMEM((GB, 128), jnp.float32),
            pltpu.VMEM_SHARED((NP, 128), jnp.float32),
            pltpu.SemaphoreType.DMA,
            pltpu.SemaphoreType.DMA,
        ],
    )
    def agg(feat, srcs, dsts, zeros, out, src_v, dst_v, g0, g1, acc, s0, s1):
        core = lax.axis_index("c")
        sub = lax.axis_index("s")
        NH = NGB // 4
        for cc in range(CPC):
            c = core * CPC + cc
            pltpu.sync_copy(zeros, g0)
            for k in range(ZG):
                pltpu.async_copy(g0, acc.at[pl.ds(sub * (ZG * GB) + k * GB, GB)], s0)
            for k in range(ZG):
                pltpu.make_async_copy(g0, acc.at[pl.ds(sub * (ZG * GB) + k * GB, GB)], s0).wait()
            plsc.subcore_barrier()

            for h in range(4):
                pltpu.sync_copy(srcs.at[c, sub, pl.ds(h * NH, NH)], src_v)
                pltpu.sync_copy(dsts.at[sub, pl.ds(h * NH, NH)], dst_v)

                pltpu.async_copy(feat.at[src_v.at[0]], g0, s0)

                def body(jj, carry):
                    j0 = 2 * jj
                    pltpu.async_copy(feat.at[src_v.at[j0 + 1]], g1, s1)
                    pltpu.make_async_copy(feat.at[src_v.at[j0]], g0, s0).wait()
                    pltpu.sync_copy(g0, acc.at[dst_v.at[j0]], add=True)

                    @pl.when(j0 + 2 < NH)
                    def _():
                        pltpu.async_copy(feat.at[src_v.at[j0 + 2]], g0, s0)

                    pltpu.make_async_copy(feat.at[src_v.at[j0 + 1]], g1, s1).wait()
                    pltpu.sync_copy(g1, acc.at[dst_v.at[j0 + 1]], add=True)
                    return carry

                lax.fori_loop(0, NH // 2, body, 0)
            plsc.subcore_barrier()

            for k in range(ZG):
                b, s = (g0, s0) if k % 2 == 0 else (g1, s1)
                r0 = sub * (ZG * GB) + k * GB
                if k >= 2:
                    rp = sub * (ZG * GB) + (k - 2) * GB
                    pltpu.make_async_copy(b, out.at[c, pl.ds(rp, GB)], s).wait()
                pltpu.sync_copy(acc.at[pl.ds(r0, GB)], b)
                pltpu.async_copy(b, out.at[c, pl.ds(r0, GB)], s)
            for k in range(ZG - 2, ZG):
                b, s = (g0, s0) if k % 2 == 0 else (g1, s1)
                r0 = sub * (ZG * GB) + k * GB
                pltpu.make_async_copy(b, out.at[c, pl.ds(r0, GB)], s).wait()
            plsc.subcore_barrier()

    return agg


_agg2 = _make_agg(2)
_agg4 = _make_agg(4)


@functools.partial(
    pl.kernel,
    mesh=_mesh,
    out_type=jax.ShapeDtypeStruct((NP, 128), jnp.float32),
    scratch_types=[
        pltpu.VMEM((NB, BLK), jnp.int32),
        pltpu.VMEM((BLK, 128), jnp.float32),
        pltpu.VMEM_SHARED((NP, 128), jnp.float32),
    ],
)
def _deg(dsts, ones, zeros, out, dst_v, buf_v, acc):
    core = lax.axis_index("c")
    sub = lax.axis_index("s")
    pltpu.sync_copy(dsts.at[sub], dst_v)
    pltpu.sync_copy(zeros, buf_v)
    for k in range(ZC):
        pltpu.sync_copy(buf_v, acc.at[pl.ds(sub * (ZC * 128) + k * 128, 128)])
    pltpu.sync_copy(ones, buf_v)
    plsc.subcore_barrier()

    def body(j, carry):
        pltpu.sync_copy(buf_v, acc.at[dst_v.at[j]], add=True)
        return carry

    lax.fori_loop(0, NB, body, 0)
    plsc.subcore_barrier()

    @pl.when(core == 0)
    def _():
        for k in range(ZC):
            r0 = sub * (ZC * 128) + k * 128
            pltpu.sync_copy(acc.at[pl.ds(r0, 128)], buf_v)
            pltpu.sync_copy(buf_v, out.at[pl.ds(r0, 128)])


def _tc_scale(x_pad, deg_col):
    def body(x_ref, deg_ref, xs_ref, dinv_ref):
        dinv = lax.rsqrt(deg_ref[...] + 1.0)
        xs_ref[0] = x_ref[...] * dinv
        dinv_ref[...] = dinv

    return pl.pallas_call(
        body,
        grid=(NR, 2),
        in_specs=[
            pl.BlockSpec((RB, 128), lambda r, c: (r, c)),
            pl.BlockSpec((RB, 1), lambda r, c: (r, 0)),
        ],
        out_specs=[
            pl.BlockSpec((1, RB, 128), lambda r, c: (c, r, 0)),
            pl.BlockSpec((RB, 1), lambda r, c: (r, 0)),
        ],
        out_shape=[
            jax.ShapeDtypeStruct((2, NP, 128), jnp.float32),
            jax.ShapeDtypeStruct((NP, 1), jnp.float32),
        ],
    )(x_pad, deg_col)


def _tc_layer1(agg1, xs, dinv, W1, b1):
    def body(agg_ref, xs_ref, dinv_ref, w_ref, b_ref, out_ref):
        dinv = dinv_ref[...]
        W = w_ref[...]
        z = jnp.zeros((RB, 512), jnp.float32)
        for i in range(2):
            u = agg_ref[i] + xs_ref[i]
            z = z + jnp.dot(u, W[i * 128:(i + 1) * 128],
                            preferred_element_type=jnp.float32)
        h = dinv * jnp.maximum(dinv * z + b_ref[...], 0.0)
        for c in range(4):
            out_ref[c] = h[:, c * 128:(c + 1) * 128]

    return pl.pallas_call(
        body,
        grid=(NR,),
        in_specs=[
            pl.BlockSpec((2, RB, 128), lambda r: (0, r, 0)),
            pl.BlockSpec((2, RB, 128), lambda r: (0, r, 0)),
            pl.BlockSpec((RB, 1), lambda r: (r, 0)),
            pl.BlockSpec((256, 512), lambda r: (0, 0)),
            pl.BlockSpec((1, 512), lambda r: (0, 0)),
        ],
        out_specs=pl.BlockSpec((4, RB, 128), lambda r: (0, r, 0)),
        out_shape=jax.ShapeDtypeStruct((4, NP, 128), jnp.float32),
    )(agg1, xs, dinv, W1, b1)


def _tc_layer2(agg2, h1s, dinv, W2, b2, W3):
    def body(agg_ref, hs_ref, dinv_ref, w2_ref, b2_ref, w3_ref, out_ref):
        dinv = dinv_ref[...]
        W2b = w2_ref[...]
        z = jnp.zeros((RB, 512), jnp.float32)
        for i in range(4):
            u = agg_ref[i] + hs_ref[i]
            z = z + jnp.dot(u, W2b[i * 128:(i + 1) * 128],
                            preferred_element_type=jnp.float32)
        h2 = jnp.maximum(dinv * z + b2_ref[...], 0.0)
        ys = dinv * jnp.dot(h2, w3_ref[...], preferred_element_type=jnp.float32)
        out_ref[0] = ys[:, :128]
        out_ref[1] = ys[:, 128:]

    return pl.pallas_call(
        body,
        grid=(NR,),
        in_specs=[
            pl.BlockSpec((4, RB, 128), lambda r: (0, r, 0)),
            pl.BlockSpec((4, RB, 128), lambda r: (0, r, 0)),
            pl.BlockSpec((RB, 1), lambda r: (r, 0)),
            pl.BlockSpec((512, 512), lambda r: (0, 0)),
            pl.BlockSpec((1, 512), lambda r: (0, 0)),
            pl.BlockSpec((512, 256), lambda r: (0, 0)),
        ],
        out_specs=pl.BlockSpec((2, RB, 128), lambda r: (0, r, 0)),
        out_shape=jax.ShapeDtypeStruct((2, NP, 128), jnp.float32),
    )(agg2, h1s, dinv, W2, b2, W3)


def _tc_layer3(agg3, ys, dinv, b3, Wc1p, bc1p, Wc2p, bc2p):
    def body(agg_ref, ys_ref, dinv_ref, b3_ref, wc1_ref, bc1_ref, wc2_ref,
             bc2_ref, out_ref):
        dinv = dinv_ref[...]
        b3v = b3_ref[...]
        Wc1 = wc1_ref[...]
        h3_0 = jnp.maximum(dinv * (agg_ref[0] + ys_ref[0]) + b3v[:, :128], 0.0)
        h3_1 = jnp.maximum(dinv * (agg_ref[1] + ys_ref[1]) + b3v[:, 128:], 0.0)
        c1 = jnp.dot(h3_0, Wc1[:128], preferred_element_type=jnp.float32)
        c1 = c1 + jnp.dot(h3_1, Wc1[128:], preferred_element_type=jnp.float32)
        c1 = jnp.maximum(c1 + bc1_ref[...], 0.0)
        out_ref[...] = jnp.dot(c1, wc2_ref[...],
                               preferred_element_type=jnp.float32) + bc2_ref[...]

    return pl.pallas_call(
        body,
        grid=(NR,),
        in_specs=[
            pl.BlockSpec((2, RB, 128), lambda r: (0, r, 0)),
            pl.BlockSpec((2, RB, 128), lambda r: (0, r, 0)),
            pl.BlockSpec((RB, 1), lambda r: (r, 0)),
            pl.BlockSpec((1, 256), lambda r: (0, 0)),
            pl.BlockSpec((256, 128), lambda r: (0, 0)),
            pl.BlockSpec((1, 128), lambda r: (0, 0)),
            pl.BlockSpec((128, 128), lambda r: (0, 0)),
            pl.BlockSpec((1, 128), lambda r: (0, 0)),
        ],
        out_specs=pl.BlockSpec((RB, 128), lambda r: (r, 0)),
        out_shape=jax.ShapeDtypeStruct((NP, 128), jnp.float32),
    )(agg3, ys, dinv, b3, Wc1p, bc1p, Wc2p, bc2p)


def kernel(x, edge_index, W1, b1, W2, b2, W3, b3, Wc1, bc1, Wc2, bc2):
    src = edge_index[0].astype(jnp.int32)
    dst = edge_index[1].astype(jnp.int32)
    src_p = jnp.concatenate([src, jnp.zeros((EP - E,), jnp.int32)])
    dst_p = jnp.concatenate([dst, jnp.full((EP - E,), N, jnp.int32)])
    src_p, dst_p = lax.sort([src_p, dst_p], num_keys=1)
    dst_slab = dst_p.reshape(NSUB, NB, BLK)
    dst_slab_g = dst_p.reshape(NSUB, NGB, GB)
    offs = (jnp.arange(4, dtype=jnp.int32) * NP)[:, None, None, None]
    src_slabs = src_p.reshape(NSUB, NGB, GB)[None] + offs
    zeros128 = jnp.zeros((128, 128), jnp.float32)
    zeros64 = jnp.zeros((GB, 128), jnp.float32)
    ones128 = jnp.ones((BLK, 128), jnp.float32)
    x_pad = jnp.pad(x, ((0, NP - N), (0, 0)))

    deg128 = _deg(dst_slab, ones128, zeros128)
    deg_col = deg128[:, :1]

    xs, dinv = _tc_scale(x_pad, deg_col)
    agg1 = _agg2(xs.reshape(2 * NP, 128), src_slabs[:2], dst_slab_g, zeros64)
    h1s = _tc_layer1(agg1, xs, dinv, W1, b1.reshape(1, 512))
    agg2 = _agg4(h1s.reshape(4 * NP, 128), src_slabs, dst_slab_g, zeros64)
    ys = _tc_layer2(agg2, h1s, dinv, W2, b2.reshape(1, 512), W3)
    agg3 = _agg2(ys.reshape(2 * NP, 128), src_slabs[:2], dst_slab_g, zeros64)

    Wc1p = jnp.pad(Wc1, ((0, 0), (0, 96)))
    bc1p = jnp.pad(bc1, (0, 96)).reshape(1, 128)
    Wc2p = jnp.pad(Wc2, ((0, 96), (0, 126)))
    bc2p = jnp.pad(bc2, (0, 126)).reshape(1, 128)
    outp = _tc_layer3(agg3, ys, dinv, b3.reshape(1, 256), Wc1p, bc1p, Wc2p, bc2p)
    return outp[:N, :2]


# speedup vs baseline: 1.5671x; 1.5671x over previous
import functools

import jax
import jax.numpy as jnp
from jax import lax
from jax.experimental import pallas as pl
from jax.experimental.pallas import tpu as pltpu
from jax.experimental.pallas import tpu_sc as plsc

N = 10000
NP = 10240
E = 160000
EP = 163840
NSUB = 16
NCORE = 2
BLK = 128
NB = EP // (NSUB * BLK)
ZC = NP // (NSUB * 128)
GB = 64
NGB = EP // (NSUB * GB)
ZG = NP // (NSUB * GB)
RB = 256
NR = NP // RB

_mesh = plsc.VectorSubcoreMesh(core_axis_name="c", subcore_axis_name="s")


def _make_agg(C):
    CPC = C // NCORE

    @functools.partial(
        pl.kernel,
        mesh=_mesh,
        out_type=jax.ShapeDtypeStruct((C, NP, 128), jnp.float32),
        scratch_types=[
            pltpu.VMEM((NGB // 2, GB), jnp.int32),
            pltpu.VMEM((NGB // 2, GB), jnp.int32),
            pltpu.VMEM((GB, 128), jnp.float32),
            pltpu.VMEM((GB, 128), jnp.float32),
            pltpu.VMEM_SHARED((NP, 128), jnp.float32),
            pltpu.SemaphoreType.DMA,
            pltpu.SemaphoreType.DMA,
        ],
    )
    def agg(feat, srcs, dsts, zeros, out, src_v, dst_v, g0, g1, acc, s0, s1):
        core = lax.axis_index("c")
        sub = lax.axis_index("s")
        NH = NGB // 2
        for cc in range(CPC):
            c = core * CPC + cc
            pltpu.sync_copy(zeros, g0)
            for k in range(ZG):
                pltpu.async_copy(g0, acc.at[pl.ds(sub * (ZG * GB) + k * GB, GB)], s0)
            for k in range(ZG):
                pltpu.make_async_copy(g0, acc.at[pl.ds(sub * (ZG * GB) + k * GB, GB)], s0).wait()
            plsc.subcore_barrier()

            for h in range(2):
                pltpu.sync_copy(srcs.at[c, sub, pl.ds(h * NH, NH)], src_v)
                pltpu.sync_copy(dsts.at[sub, pl.ds(h * NH, NH)], dst_v)

                pltpu.async_copy(feat.at[src_v.at[0]], g0, s0)

                def body(jj, carry):
                    j0 = 2 * jj
                    pltpu.async_copy(feat.at[src_v.at[j0 + 1]], g1, s1)
                    pltpu.make_async_copy(feat.at[src_v.at[j0]], g0, s0).wait()
                    pltpu.sync_copy(g0, acc.at[dst_v.at[j0]], add=True)

                    @pl.when(j0 + 2 < NH)
                    def _():
                        pltpu.async_copy(feat.at[src_v.at[j0 + 2]], g0, s0)

                    pltpu.make_async_copy(feat.at[src_v.at[j0 + 1]], g1, s1).wait()
                    pltpu.sync_copy(g1, acc.at[dst_v.at[j0 + 1]], add=True)
                    return carry

                lax.fori_loop(0, NH // 2, body, 0)
            plsc.subcore_barrier()

            for k in range(ZG):
                b, s = (g0, s0) if k % 2 == 0 else (g1, s1)
                r0 = sub * (ZG * GB) + k * GB
                if k >= 2:
                    rp = sub * (ZG * GB) + (k - 2) * GB
                    pltpu.make_async_copy(b, out.at[c, pl.ds(rp, GB)], s).wait()
                pltpu.sync_copy(acc.at[pl.ds(r0, GB)], b)
                pltpu.async_copy(b, out.at[c, pl.ds(r0, GB)], s)
            for k in range(ZG - 2, ZG):
                b, s = (g0, s0) if k % 2 == 0 else (g1, s1)
                r0 = sub * (ZG * GB) + k * GB
                pltpu.make_async_copy(b, out.at[c, pl.ds(r0, GB)], s).wait()
            plsc.subcore_barrier()

    return agg


_agg2 = _make_agg(2)
_agg4 = _make_agg(4)


@functools.partial(
    pl.kernel,
    mesh=_mesh,
    out_type=jax.ShapeDtypeStruct((NP, 128), jnp.float32),
    scratch_types=[
        pltpu.VMEM((NB, BLK), jnp.int32),
        pltpu.VMEM((BLK, 128), jnp.float32),
        pltpu.VMEM_SHARED((NP, 128), jnp.float32),
        pltpu.SemaphoreType.DMA,
    ],
)
def _deg(dsts, ones, zeros, out, dst_v, buf_v, acc, dsem):
    core = lax.axis_index("c")
    sub = lax.axis_index("s")
    pltpu.sync_copy(dsts.at[sub], dst_v)
    pltpu.sync_copy(zeros, buf_v)
    for k in range(ZC):
        pltpu.sync_copy(buf_v, acc.at[pl.ds(sub * (ZC * 128) + k * 128, 128)])
    pltpu.sync_copy(ones, buf_v)
    plsc.subcore_barrier()

    def body(j, carry):
        pltpu.async_copy(buf_v, acc.at[dst_v.at[j]], dsem, add=True)
        return carry

    lax.fori_loop(0, NB, body, 0)

    def drain(j, carry):
        pltpu.make_async_copy(buf_v, acc.at[dst_v.at[0]], dsem).wait()
        return carry

    lax.fori_loop(0, NB, drain, 0)
    plsc.subcore_barrier()

    @pl.when(core == 0)
    def _():
        for k in range(ZC):
            r0 = sub * (ZC * 128) + k * 128
            pltpu.sync_copy(acc.at[pl.ds(r0, 128)], buf_v)
            pltpu.sync_copy(buf_v, out.at[pl.ds(r0, 128)])


def _tc_scale(x_pad, deg_col):
    def body(x_ref, deg_ref, xs_ref, dinv_ref):
        dinv = lax.rsqrt(deg_ref[...] + 1.0)
        xs_ref[0] = x_ref[...] * dinv
        dinv_ref[...] = dinv

    return pl.pallas_call(
        body,
        grid=(NR, 2),
        in_specs=[
            pl.BlockSpec((RB, 128), lambda r, c: (r, c)),
            pl.BlockSpec((RB, 1), lambda r, c: (r, 0)),
        ],
        out_specs=[
            pl.BlockSpec((1, RB, 128), lambda r, c: (c, r, 0)),
            pl.BlockSpec((RB, 1), lambda r, c: (r, 0)),
        ],
        out_shape=[
            jax.ShapeDtypeStruct((2, NP, 128), jnp.float32),
            jax.ShapeDtypeStruct((NP, 1), jnp.float32),
        ],
    )(x_pad, deg_col)


def _tc_layer1(agg1, xs, dinv, W1, b1):
    def body(agg_ref, xs_ref, dinv_ref, w_ref, b_ref, out_ref):
        dinv = dinv_ref[...]
        W = w_ref[...]
        z = jnp.zeros((RB, 512), jnp.float32)
        for i in range(2):
            u = agg_ref[i] + xs_ref[i]
            z = z + jnp.dot(u, W[i * 128:(i + 1) * 128],
                            preferred_element_type=jnp.float32)
        h = dinv * jnp.maximum(dinv * z + b_ref[...], 0.0)
        for c in range(4):
            out_ref[c] = h[:, c * 128:(c + 1) * 128]

    return pl.pallas_call(
        body,
        grid=(NR,),
        in_specs=[
            pl.BlockSpec((2, RB, 128), lambda r: (0, r, 0)),
            pl.BlockSpec((2, RB, 128), lambda r: (0, r, 0)),
            pl.BlockSpec((RB, 1), lambda r: (r, 0)),
            pl.BlockSpec((256, 512), lambda r: (0, 0)),
            pl.BlockSpec((1, 512), lambda r: (0, 0)),
        ],
        out_specs=pl.BlockSpec((4, RB, 128), lambda r: (0, r, 0)),
        out_shape=jax.ShapeDtypeStruct((4, NP, 128), jnp.float32),
    )(agg1, xs, dinv, W1, b1)


def _tc_layer2(agg2, h1s, dinv, W2, b2, W3):
    def body(agg_ref, hs_ref, dinv_ref, w2_ref, b2_ref, w3_ref, out_ref):
        dinv = dinv_ref[...]
        W2b = w2_ref[...]
        z = jnp.zeros((RB, 512), jnp.float32)
        for i in range(4):
            u = agg_ref[i] + hs_ref[i]
            z = z + jnp.dot(u, W2b[i * 128:(i + 1) * 128],
                            preferred_element_type=jnp.float32)
        h2 = jnp.maximum(dinv * z + b2_ref[...], 0.0)
        ys = dinv * jnp.dot(h2, w3_ref[...], preferred_element_type=jnp.float32)
        out_ref[0] = ys[:, :128]
        out_ref[1] = ys[:, 128:]

    return pl.pallas_call(
        body,
        grid=(NR,),
        in_specs=[
            pl.BlockSpec((4, RB, 128), lambda r: (0, r, 0)),
            pl.BlockSpec((4, RB, 128), lambda r: (0, r, 0)),
            pl.BlockSpec((RB, 1), lambda r: (r, 0)),
            pl.BlockSpec((512, 512), lambda r: (0, 0)),
            pl.BlockSpec((1, 512), lambda r: (0, 0)),
            pl.BlockSpec((512, 256), lambda r: (0, 0)),
        ],
        out_specs=pl.BlockSpec((2, RB, 128), lambda r: (0, r, 0)),
        out_shape=jax.ShapeDtypeStruct((2, NP, 128), jnp.float32),
    )(agg2, h1s, dinv, W2, b2, W3)


def _tc_layer3(agg3, ys, dinv, b3, Wc1p, bc1p, Wc2p, bc2p):
    def body(agg_ref, ys_ref, dinv_ref, b3_ref, wc1_ref, bc1_ref, wc2_ref,
             bc2_ref, out_ref):
        dinv = dinv_ref[...]
        b3v = b3_ref[...]
        Wc1 = wc1_ref[...]
        h3_0 = jnp.maximum(dinv * (agg_ref[0] + ys_ref[0]) + b3v[:, :128], 0.0)
        h3_1 = jnp.maximum(dinv * (agg_ref[1] + ys_ref[1]) + b3v[:, 128:], 0.0)
        c1 = jnp.dot(h3_0, Wc1[:128], preferred_element_type=jnp.float32)
        c1 = c1 + jnp.dot(h3_1, Wc1[128:], preferred_element_type=jnp.float32)
        c1 = jnp.maximum(c1 + bc1_ref[...], 0.0)
        out_ref[...] = jnp.dot(c1, wc2_ref[...],
                               preferred_element_type=jnp.float32) + bc2_ref[...]

    return pl.pallas_call(
        body,
        grid=(NR,),
        in_specs=[
            pl.BlockSpec((2, RB, 128), lambda r: (0, r, 0)),
            pl.BlockSpec((2, RB, 128), lambda r: (0, r, 0)),
            pl.BlockSpec((RB, 1), lambda r: (r, 0)),
            pl.BlockSpec((1, 256), lambda r: (0, 0)),
            pl.BlockSpec((256, 128), lambda r: (0, 0)),
            pl.BlockSpec((1, 128), lambda r: (0, 0)),
            pl.BlockSpec((128, 128), lambda r: (0, 0)),
            pl.BlockSpec((1, 128), lambda r: (0, 0)),
        ],
        out_specs=pl.BlockSpec((RB, 128), lambda r: (r, 0)),
        out_shape=jax.ShapeDtypeStruct((NP, 128), jnp.float32),
    )(agg3, ys, dinv, b3, Wc1p, bc1p, Wc2p, bc2p)


def kernel(x, edge_index, W1, b1, W2, b2, W3, b3, Wc1, bc1, Wc2, bc2):
    src = edge_index[0].astype(jnp.int32)
    dst = edge_index[1].astype(jnp.int32)
    src_p = jnp.concatenate([src, jnp.zeros((EP - E,), jnp.int32)])
    dst_p = jnp.concatenate([dst, jnp.full((EP - E,), N, jnp.int32)])
    dst_slab = dst_p.reshape(NSUB, NB, BLK)
    dst_slab_g = dst_p.reshape(NSUB, NGB, GB)
    offs = (jnp.arange(4, dtype=jnp.int32) * NP)[:, None, None, None]
    src_slabs = src_p.reshape(NSUB, NGB, GB)[None] + offs
    zeros128 = jnp.zeros((128, 128), jnp.float32)
    zeros64 = jnp.zeros((GB, 128), jnp.float32)
    ones128 = jnp.ones((BLK, 128), jnp.float32)
    x_pad = jnp.pad(x, ((0, NP - N), (0, 0)))

    deg128 = _deg(dst_slab, ones128, zeros128)
    deg_col = deg128[:, :1]

    xs, dinv = _tc_scale(x_pad, deg_col)
    agg1 = _agg2(xs.reshape(2 * NP, 128), src_slabs[:2], dst_slab_g, zeros64)
    h1s = _tc_layer1(agg1, xs, dinv, W1, b1.reshape(1, 512))
    agg2 = _agg4(h1s.reshape(4 * NP, 128), src_slabs, dst_slab_g, zeros64)
    ys = _tc_layer2(agg2, h1s, dinv, W2, b2.reshape(1, 512), W3)
    agg3 = _agg2(ys.reshape(2 * NP, 128), src_slabs[:2], dst_slab_g, zeros64)

    Wc1p = jnp.pad(Wc1, ((0, 0), (0, 96)))
    bc1p = jnp.pad(bc1, (0, 96)).reshape(1, 128)
    Wc2p = jnp.pad(Wc2, ((0, 96), (0, 126)))
    bc2p = jnp.pad(bc2, (0, 126)).reshape(1, 128)
    outp = _tc_layer3(agg3, ys, dinv, b3.reshape(1, 256), Wc1p, bc1p, Wc2p, bc2p)
    return outp[:N, :2]


# speedup vs baseline: 1.5681x; 1.0007x over previous
import functools

import jax
import jax.numpy as jnp
from jax import lax
from jax.experimental import pallas as pl
from jax.experimental.pallas import tpu as pltpu
from jax.experimental.pallas import tpu_sc as plsc

N = 10000
NP = 10240
E = 160000
EP = 163840
NSUB = 16
NCORE = 2
BLK = 128
NB = EP // (NSUB * BLK)
ZC = NP // (NSUB * 128)
GB = 64
NGB = EP // (NSUB * GB)
ZG = NP // (NSUB * GB)
RB = 256
NR = NP // RB

_mesh = plsc.VectorSubcoreMesh(core_axis_name="c", subcore_axis_name="s")


def _make_agg(C):
    CPC = C // NCORE

    @functools.partial(
        pl.kernel,
        mesh=_mesh,
        out_type=jax.ShapeDtypeStruct((C, NP, 128), jnp.float32),
        scratch_types=[
            pltpu.VMEM((NGB // 2, GB), jnp.int32),
            pltpu.VMEM((NGB // 2, GB), jnp.int32),
            pltpu.VMEM((GB, 128), jnp.float32),
            pltpu.VMEM((GB, 128), jnp.float32),
            pltpu.VMEM_SHARED((NP, 128), jnp.float32),
            pltpu.SemaphoreType.DMA,
            pltpu.SemaphoreType.DMA,
        ],
    )
    def agg(feat, srcs, dsts, zeros, out, src_v, dst_v, g0, g1, acc, s0, s1):
        core = lax.axis_index("c")
        sub = lax.axis_index("s")
        NH = NGB // 2
        for cc in range(CPC):
            c = core * CPC + cc
            pltpu.sync_copy(zeros, g0)
            for k in range(ZG):
                pltpu.async_copy(g0, acc.at[pl.ds(sub * (ZG * GB) + k * GB, GB)], s0)
            for k in range(ZG):
                pltpu.make_async_copy(g0, acc.at[pl.ds(sub * (ZG * GB) + k * GB, GB)], s0).wait()
            plsc.subcore_barrier()

            for h in range(2):
                pltpu.sync_copy(srcs.at[c, sub, pl.ds(h * NH, NH)], src_v)
                pltpu.sync_copy(dsts.at[sub, pl.ds(h * NH, NH)], dst_v)

                pltpu.async_copy(feat.at[src_v.at[0]], g0, s0)

                def body(jj, carry):
                    j0 = 2 * jj
                    pltpu.async_copy(feat.at[src_v.at[j0 + 1]], g1, s1)
                    pltpu.make_async_copy(feat.at[src_v.at[j0]], g0, s0).wait()
                    pltpu.sync_copy(g0, acc.at[dst_v.at[j0]], add=True)

                    @pl.when(j0 + 2 < NH)
                    def _():
                        pltpu.async_copy(feat.at[src_v.at[j0 + 2]], g0, s0)

                    pltpu.make_async_copy(feat.at[src_v.at[j0 + 1]], g1, s1).wait()
                    pltpu.sync_copy(g1, acc.at[dst_v.at[j0 + 1]], add=True)
                    return carry

                lax.fori_loop(0, NH // 2, body, 0)
            plsc.subcore_barrier()

            for k in range(ZG):
                b, s = (g0, s0) if k % 2 == 0 else (g1, s1)
                r0 = sub * (ZG * GB) + k * GB
                if k >= 2:
                    rp = sub * (ZG * GB) + (k - 2) * GB
                    pltpu.make_async_copy(b, out.at[c, pl.ds(rp, GB)], s).wait()
                pltpu.sync_copy(acc.at[pl.ds(r0, GB)], b)
                pltpu.async_copy(b, out.at[c, pl.ds(r0, GB)], s)
            for k in range(ZG - 2, ZG):
                b, s = (g0, s0) if k % 2 == 0 else (g1, s1)
                r0 = sub * (ZG * GB) + k * GB
                pltpu.make_async_copy(b, out.at[c, pl.ds(r0, GB)], s).wait()
            plsc.subcore_barrier()

    return agg


_agg2 = _make_agg(2)
_agg4 = _make_agg(4)


@functools.partial(
    pl.kernel,
    mesh=_mesh,
    out_type=jax.ShapeDtypeStruct((NP, 128), jnp.float32),
    scratch_types=[
        pltpu.VMEM((NB, BLK), jnp.int32),
        pltpu.VMEM((BLK, 128), jnp.float32),
        pltpu.VMEM_SHARED((NP, 128), jnp.float32),
        pltpu.SemaphoreType.DMA,
    ],
)
def _deg(dsts, ones, zeros, out, dst_v, buf_v, acc, dsem):
    core = lax.axis_index("c")
    sub = lax.axis_index("s")
    pltpu.sync_copy(dsts.at[sub], dst_v)
    pltpu.sync_copy(zeros, buf_v)
    for k in range(ZC):
        pltpu.sync_copy(buf_v, acc.at[pl.ds(sub * (ZC * 128) + k * 128, 128)])
    pltpu.sync_copy(ones, buf_v)
    plsc.subcore_barrier()

    def body(j, carry):
        pltpu.async_copy(buf_v, acc.at[dst_v.at[j]], dsem, add=True)
        return carry

    lax.fori_loop(0, NB, body, 0)

    def drain(j, carry):
        pltpu.make_async_copy(buf_v, acc.at[dst_v.at[0]], dsem).wait()
        return carry

    lax.fori_loop(0, NB, drain, 0)
    plsc.subcore_barrier()

    @pl.when(core == 0)
    def _():
        for k in range(ZC):
            r0 = sub * (ZC * 128) + k * 128
            pltpu.sync_copy(acc.at[pl.ds(r0, 128)], buf_v)
            pltpu.sync_copy(buf_v, out.at[pl.ds(r0, 128)])


def _tc_scale(x_pad, deg_col):
    def body(x_ref, deg_ref, xs_ref, dinv_ref):
        dinv = lax.rsqrt(deg_ref[...] + 1.0)
        xs_ref[0] = x_ref[...] * dinv
        dinv_ref[...] = dinv

    return pl.pallas_call(
        body,
        grid=(NR, 2),
        in_specs=[
            pl.BlockSpec((RB, 128), lambda r, c: (r, c)),
            pl.BlockSpec((RB, 1), lambda r, c: (r, 0)),
        ],
        out_specs=[
            pl.BlockSpec((1, RB, 128), lambda r, c: (c, r, 0)),
            pl.BlockSpec((RB, 1), lambda r, c: (r, 0)),
        ],
        out_shape=[
            jax.ShapeDtypeStruct((2, NP, 128), jnp.float32),
            jax.ShapeDtypeStruct((NP, 1), jnp.float32),
        ],
    )(x_pad, deg_col)


def _tc_layer1(agg1, xs, dinv, W1, b1):
    def body(agg_ref, xs_ref, dinv_ref, w_ref, b_ref, out_ref):
        dinv = dinv_ref[...]
        W = w_ref[...]
        z = jnp.zeros((RB, 512), jnp.float32)
        for i in range(2):
            u = agg_ref[i] + xs_ref[i]
            z = z + jnp.dot(u, W[i * 128:(i + 1) * 128],
                            preferred_element_type=jnp.float32)
        h = dinv * jnp.maximum(dinv * z + b_ref[...], 0.0)
        for c in range(4):
            out_ref[c] = h[:, c * 128:(c + 1) * 128]

    return pl.pallas_call(
        body,
        grid=(NR,),
        in_specs=[
            pl.BlockSpec((2, RB, 128), lambda r: (0, r, 0)),
            pl.BlockSpec((2, RB, 128), lambda r: (0, r, 0)),
            pl.BlockSpec((RB, 1), lambda r: (r, 0)),
            pl.BlockSpec((256, 512), lambda r: (0, 0)),
            pl.BlockSpec((1, 512), lambda r: (0, 0)),
        ],
        out_specs=pl.BlockSpec((4, RB, 128), lambda r: (0, r, 0)),
        out_shape=jax.ShapeDtypeStruct((4, NP, 128), jnp.float32),
    )(agg1, xs, dinv, W1, b1)


def _tc_layer2(agg2, h1s, dinv, W2, b2, W3):
    def body(agg_ref, hs_ref, dinv_ref, w2_ref, b2_ref, w3_ref, out_ref):
        dinv = dinv_ref[...]
        W2b = w2_ref[...]
        z = jnp.zeros((RB, 512), jnp.float32)
        for i in range(4):
            u = agg_ref[i] + hs_ref[i]
            z = z + jnp.dot(u, W2b[i * 128:(i + 1) * 128],
                            preferred_element_type=jnp.float32)
        h2 = jnp.maximum(dinv * z + b2_ref[...], 0.0)
        ys = dinv * jnp.dot(h2, w3_ref[...], preferred_element_type=jnp.float32)
        out_ref[0] = ys[:, :128]
        out_ref[1] = ys[:, 128:]

    return pl.pallas_call(
        body,
        grid=(NR,),
        in_specs=[
            pl.BlockSpec((4, RB, 128), lambda r: (0, r, 0)),
            pl.BlockSpec((4, RB, 128), lambda r: (0, r, 0)),
            pl.BlockSpec((RB, 1), lambda r: (r, 0)),
            pl.BlockSpec((512, 512), lambda r: (0, 0)),
            pl.BlockSpec((1, 512), lambda r: (0, 0)),
            pl.BlockSpec((512, 256), lambda r: (0, 0)),
        ],
        out_specs=pl.BlockSpec((2, RB, 128), lambda r: (0, r, 0)),
        out_shape=jax.ShapeDtypeStruct((2, NP, 128), jnp.float32),
    )(agg2, h1s, dinv, W2, b2, W3)


def _tc_layer3(agg3, ys, dinv, b3, Wc1p, bc1p, Wc2p, bc2p):
    def body(agg_ref, ys_ref, dinv_ref, b3_ref, wc1_ref, bc1_ref, wc2_ref,
             bc2_ref, out_ref):
        dinv = dinv_ref[...]
        b3v = b3_ref[...]
        Wc1 = wc1_ref[...]
        h3_0 = jnp.maximum(dinv * (agg_ref[0] + ys_ref[0]) + b3v[:, :128], 0.0)
        h3_1 = jnp.maximum(dinv * (agg_ref[1] + ys_ref[1]) + b3v[:, 128:], 0.0)
        c1 = jnp.dot(h3_0, Wc1[:128], preferred_element_type=jnp.float32)
        c1 = c1 + jnp.dot(h3_1, Wc1[128:], preferred_element_type=jnp.float32)
        c1 = jnp.maximum(c1 + bc1_ref[...], 0.0)
        out_ref[...] = jnp.dot(c1, wc2_ref[...],
                               preferred_element_type=jnp.float32) + bc2_ref[...]

    return pl.pallas_call(
        body,
        grid=(NR,),
        in_specs=[
            pl.BlockSpec((2, RB, 128), lambda r: (0, r, 0)),
            pl.BlockSpec((2, RB, 128), lambda r: (0, r, 0)),
            pl.BlockSpec((RB, 1), lambda r: (r, 0)),
            pl.BlockSpec((1, 256), lambda r: (0, 0)),
            pl.BlockSpec((256, 128), lambda r: (0, 0)),
            pl.BlockSpec((1, 128), lambda r: (0, 0)),
            pl.BlockSpec((128, 128), lambda r: (0, 0)),
            pl.BlockSpec((1, 128), lambda r: (0, 0)),
        ],
        out_specs=pl.BlockSpec((RB, 128), lambda r: (r, 0)),
        out_shape=jax.ShapeDtypeStruct((NP, 128), jnp.float32),
    )(agg3, ys, dinv, b3, Wc1p, bc1p, Wc2p, bc2p)


def kernel(x, edge_index, W1, b1, W2, b2, W3, b3, Wc1, bc1, Wc2, bc2):
    src = edge_index[0].astype(jnp.int32)
    dst = edge_index[1].astype(jnp.int32)
    src_p = jnp.concatenate([src, jnp.zeros((EP - E,), jnp.int32)])
    pad_dst = N + jnp.arange(EP - E, dtype=jnp.int32) % (NP - N)
    dst_p = jnp.concatenate([dst, pad_dst])
    dst_slab = dst_p.reshape(NSUB, NB, BLK)
    dst_slab_g = dst_p.reshape(NSUB, NGB, GB)
    offs = (jnp.arange(4, dtype=jnp.int32) * NP)[:, None, None, None]
    src_slabs = src_p.reshape(NSUB, NGB, GB)[None] + offs
    zeros128 = jnp.zeros((128, 128), jnp.float32)
    zeros64 = jnp.zeros((GB, 128), jnp.float32)
    ones128 = jnp.ones((BLK, 128), jnp.float32)
    x_pad = jnp.pad(x, ((0, NP - N), (0, 0)))

    deg128 = _deg(dst_slab, ones128, zeros128)
    deg_col = deg128[:, :1]

    xs, dinv = _tc_scale(x_pad, deg_col)
    agg1 = _agg2(xs.reshape(2 * NP, 128), src_slabs[:2], dst_slab_g, zeros64)
    h1s = _tc_layer1(agg1, xs, dinv, W1, b1.reshape(1, 512))
    agg2 = _agg4(h1s.reshape(4 * NP, 128), src_slabs, dst_slab_g, zeros64)
    ys = _tc_layer2(agg2, h1s, dinv, W2, b2.reshape(1, 512), W3)
    agg3 = _agg2(ys.reshape(2 * NP, 128), src_slabs[:2], dst_slab_g, zeros64)

    Wc1p = jnp.pad(Wc1, ((0, 0), (0, 96)))
    bc1p = jnp.pad(bc1, (0, 96)).reshape(1, 128)
    Wc2p = jnp.pad(Wc2, ((0, 96), (0, 126)))
    bc2p = jnp.pad(bc2, (0, 126)).reshape(1, 128)
    outp = _tc_layer3(agg3, ys, dinv, b3.reshape(1, 256), Wc1p, bc1p, Wc2p, bc2p)
    return outp[:N, :2]
